# baseline (speedup 1.0000x reference)
import jax
import jax.numpy as jnp
from jax.experimental import pallas as pl
from jax.experimental.pallas import tpu as pltpu

_K = 8
_LATENT = 128
_COND = 512
_HID = 128
_STEPS = 4
_TA = 16
_DA = 32
_BT = 2048


def _dot16(a16, b16):
    return jnp.dot(a16, b16, preferred_element_type=jnp.float32)


def _dotb(a16, b16):
    return jnp.dot(a16, b16,
                   preferred_element_type=jnp.float32).astype(jnp.bfloat16)


def _silu(x):
    return (0.5 * x) * (jnp.tanh(0.5 * x) + 1.0)


def _moe_body(fo_ref, pe_ref, sl_ref, gate_ref, Wc_ref, bc_ref, W1_ref,
              b1_ref, W2_ref, b2_ref, W3_ref, b3_ref, Wd_ref, bd_ref,
              out_ref, F_s, bfold_s, W1c_s, W1u_s, tau_s, W2_s, W3_s, Wd_s, b3w_s,
              b2_s):
    bf16 = jnp.bfloat16
    d_fo = fo_ref.shape[1]
    d_pe = pe_ref.shape[1]

    @pl.when(pl.program_id(0) == 0)
    def _init():
        Wd_s[...] = Wd_ref[...].astype(bf16)
        b3w_s[...] = b3_ref[...].astype(bf16)
        b2_s[...] = b2_ref[...].astype(bf16)
        for k in range(_K):
            cols = slice(k * _HID, (k + 1) * _HID)
            W1c_s[:, cols] = W1_ref[k, _LATENT:-1, :].astype(bf16)
            W1u_s[:, cols] = W1_ref[k, :_LATENT, :].astype(bf16)
            tau_s[:, cols] = W1_ref[k, -1:, :].astype(bf16)
        F_s[...] = jnp.dot(Wc_ref[...].astype(bf16), W1c_s[...],
                           preferred_element_type=jnp.float32).astype(bf16)
        bfold_s[...] = (jnp.dot(bc_ref[...].astype(bf16), W1c_s[...],
                                preferred_element_type=jnp.float32)
                        + b1_ref[...]).astype(bf16)
        for k in range(_K):
            W2_s[k] = W2_ref[k].astype(bf16)
            W3_s[k] = W3_ref[k].astype(bf16)

    gate = gate_ref[...]
    cp16 = (_dot16(fo_ref[...].astype(bf16), F_s[:d_fo])
            + _dot16(pe_ref[...].astype(bf16), F_s[d_fo:d_fo + d_pe])
            + _dot16(sl_ref[...].astype(bf16), F_s[d_fo + d_pe:])
            + bfold_s[...]).astype(bf16)
    gb3 = _dot16(gate.astype(bf16), b3w_s[...])
    g16 = (0.5 * gate).astype(bf16)
    tau16 = tau_s[...]
    half = jnp.bfloat16(0.5)
    one = jnp.bfloat16(1.0)

    dt = 1.0 / _STEPS
    u = None
    u16 = None
    for i in range(_STEPS):
        p16 = cp16 if i == 0 else (
            _dotb(u16, W1u_s[...]) + cp16 + jnp.bfloat16(i * dt) * tau16)
        h116 = _silu(p16)
        v = gb3
        for k in range(_K):
            a216 = _dotb(h116[:, k * _HID:(k + 1) * _HID], W2_s[k]) + b2_s[k]
            h2g = (g16[:, k:k + 1] * a216) * (jnp.tanh(half * a216) + one)
            v = v + _dot16(h2g, W3_s[k])
        u = dt * v if i == 0 else u + dt * v
        u16 = u.astype(bf16)

    out_ref[...] = _dot16(u16, Wd_s[...]) + bd_ref[...]


@jax.jit
def kernel(fused_obs, phase_embed, skill_latent, p_hat, beta, Wc, bc, W1, b1,
           W2, b2, W3, b3, Wd, bd):
    del beta
    b = fused_obs.shape[0]
    d_fo = fused_obs.shape[1]
    d_pe = phase_embed.shape[1]
    d_sl = skill_latent.shape[1]
    cond_in = d_fo + d_pe + d_sl
    ein = W1.shape[1]
    out_dim = Wd.shape[1]
    bf16 = jnp.bfloat16

    grid = (b // _BT,)
    full = lambda *s: pl.BlockSpec(s, lambda i: (0,) * len(s))

    out = pl.pallas_call(
        _moe_body,
        grid=grid,
        in_specs=[
            pl.BlockSpec((_BT, d_fo), lambda i: (i, 0)),
            pl.BlockSpec((_BT, d_pe), lambda i: (i, 0)),
            pl.BlockSpec((_BT, d_sl), lambda i: (i, 0)),
            pl.BlockSpec((_BT, _K), lambda i: (i, 0)),
            full(cond_in, Wc.shape[1]),
            full(1, bc.shape[0]),
            full(_K, ein, _HID),
            full(1, _K * _HID),
            full(_K, _HID, _HID),
            full(_K, 1, _HID),
            full(_K, _HID, _LATENT),
            full(_K, _LATENT),
            full(_LATENT, out_dim),
            full(1, out_dim),
        ],
        out_specs=pl.BlockSpec((_BT, out_dim), lambda i: (i, 0)),
        out_shape=jax.ShapeDtypeStruct((b, out_dim), jnp.float32),
        scratch_shapes=[
            pltpu.VMEM((cond_in, _K * _HID), bf16),
            pltpu.VMEM((1, _K * _HID), bf16),
            pltpu.VMEM((_COND, _K * _HID), bf16),
            pltpu.VMEM((_LATENT, _K * _HID), bf16),
            pltpu.VMEM((1, _K * _HID), bf16),
            pltpu.VMEM((_K, _HID, _HID), bf16),
            pltpu.VMEM((_K, _HID, _LATENT), bf16),
            pltpu.VMEM((_LATENT, out_dim), bf16),
            pltpu.VMEM((_K, _LATENT), bf16),
            pltpu.VMEM((_K, 1, _HID), bf16),
        ],
        compiler_params=pltpu.CompilerParams(
            dimension_semantics=("arbitrary",)),
    )(fused_obs, phase_embed, skill_latent, p_hat, Wc, bc.reshape(1, -1),
      W1, b1.reshape(1, _K * _HID), W2, b2.reshape(_K, 1, _HID), W3, b3, Wd,
      bd.reshape(1, -1))

    return out.reshape(b, _TA, _DA)


# speedup vs baseline: 1.0022x; 1.0022x over previous
import jax
import jax.numpy as jnp
from jax.experimental import pallas as pl
from jax.experimental.pallas import tpu as pltpu

_K = 8
_LATENT = 128
_COND = 512
_HID = 128
_STEPS = 4
_TA = 16
_DA = 32
_BT = 2048


def _dot16(a16, b16):
    return jnp.dot(a16, b16, preferred_element_type=jnp.float32)


def _silu(x):
    return (0.5 * x) * (jnp.tanh(0.5 * x) + 1.0)


def _moe_body(fo_ref, pe_ref, sl_ref, gate_ref, Wc_ref, bc_ref, W1_ref,
              b1_ref, W2_ref, b2_ref, W3_ref, b3_ref, Wd_ref, bd_ref,
              out_ref, Wc_s, W1c_s, W1u_s, tau_s, W2_s, W3_s, Wd_s, b3w_s):
    bf16 = jnp.bfloat16
    d_fo = fo_ref.shape[1]
    d_pe = pe_ref.shape[1]

    @pl.when(pl.program_id(0) == 0)
    def _init():
        Wc_s[...] = Wc_ref[...].astype(bf16)
        Wd_s[...] = Wd_ref[...].astype(bf16)
        b3w_s[...] = b3_ref[...].astype(bf16)
        for k in range(_K):
            cols = slice(k * _HID, (k + 1) * _HID)
            W1c_s[:, cols] = W1_ref[k, _LATENT:-1, :].astype(bf16)
            W1u_s[:, cols] = W1_ref[k, :_LATENT, :].astype(bf16)
            tau_s[:, cols] = W1_ref[k, -1:, :]
            W2_s[k] = W2_ref[k].astype(bf16)
            W3_s[k] = W3_ref[k].astype(bf16)

    gate = gate_ref[...]
    cond = (_dot16(fo_ref[...].astype(bf16), Wc_s[:d_fo])
            + _dot16(pe_ref[...].astype(bf16), Wc_s[d_fo:d_fo + d_pe])
            + _dot16(sl_ref[...].astype(bf16), Wc_s[d_fo + d_pe:])
            + bc_ref[...])
    cp = _dot16(cond.astype(bf16), W1c_s[...]) + b1_ref[...]
    gb3 = _dot16(gate.astype(bf16), b3w_s[...])
    g16 = (0.5 * gate).astype(bf16)
    taurow = tau_s[...]

    dt = 1.0 / _STEPS
    u = None
    u16 = None
    for i in range(_STEPS):
        pre = cp if i == 0 else (
            _dot16(u16, W1u_s[...]) + cp + (i * dt) * taurow)
        p16 = pre.astype(bf16)
        h116 = _silu(p16)
        v = gb3
        for k in range(_K):
            a2 = _dot16(h116[:, k * _HID:(k + 1) * _HID], W2_s[k]) + b2_ref[k]
            a216 = a2.astype(bf16)
            h2g = (g16[:, k:k + 1] * a216) * (jnp.tanh(jnp.bfloat16(0.5) * a216)
                                              + jnp.bfloat16(1.0))
            v = v + _dot16(h2g, W3_s[k])
        u = dt * v if i == 0 else u + dt * v
        u16 = u.astype(bf16)

    out_ref[...] = _dot16(u16, Wd_s[...]) + bd_ref[...]


@jax.jit
def kernel(fused_obs, phase_embed, skill_latent, p_hat, beta, Wc, bc, W1, b1,
           W2, b2, W3, b3, Wd, bd):
    del beta
    b = fused_obs.shape[0]
    d_fo = fused_obs.shape[1]
    d_pe = phase_embed.shape[1]
    d_sl = skill_latent.shape[1]
    cond_in = d_fo + d_pe + d_sl
    ein = W1.shape[1]
    out_dim = Wd.shape[1]
    bf16 = jnp.bfloat16

    grid = (b // _BT,)
    full = lambda *s: pl.BlockSpec(s, lambda i: (0,) * len(s))

    out = pl.pallas_call(
        _moe_body,
        grid=grid,
        in_specs=[
            pl.BlockSpec((_BT, d_fo), lambda i: (i, 0)),
            pl.BlockSpec((_BT, d_pe), lambda i: (i, 0)),
            pl.BlockSpec((_BT, d_sl), lambda i: (i, 0)),
            pl.BlockSpec((_BT, _K), lambda i: (i, 0)),
            full(cond_in, Wc.shape[1]),
            full(1, bc.shape[0]),
            full(_K, ein, _HID),
            full(1, _K * _HID),
            full(_K, _HID, _HID),
            full(_K, 1, _HID),
            full(_K, _HID, _LATENT),
            full(_K, _LATENT),
            full(_LATENT, out_dim),
            full(1, out_dim),
        ],
        out_specs=pl.BlockSpec((_BT, out_dim), lambda i: (i, 0)),
        out_shape=jax.ShapeDtypeStruct((b, out_dim), jnp.float32),
        scratch_shapes=[
            pltpu.VMEM((cond_in, Wc.shape[1]), bf16),
            pltpu.VMEM((_COND, _K * _HID), bf16),
            pltpu.VMEM((_LATENT, _K * _HID), bf16),
            pltpu.VMEM((1, _K * _HID), jnp.float32),
            pltpu.VMEM((_K, _HID, _HID), bf16),
            pltpu.VMEM((_K, _HID, _LATENT), bf16),
            pltpu.VMEM((_LATENT, out_dim), bf16),
            pltpu.VMEM((_K, _LATENT), bf16),
        ],
        compiler_params=pltpu.CompilerParams(
            dimension_semantics=("arbitrary",)),
    )(fused_obs, phase_embed, skill_latent, p_hat, Wc, bc.reshape(1, -1),
      W1, b1.reshape(1, _K * _HID), W2, b2.reshape(_K, 1, _HID), W3, b3, Wd,
      bd.reshape(1, -1))

    return out.reshape(b, _TA, _DA)
